# baseline (speedup 1.0000x reference)
import functools

import jax
import jax.numpy as jnp
from jax import lax
from jax.experimental import pallas as pl
from jax.experimental.pallas import tpu as pltpu
from jax.experimental.pallas import tpu_sc as plsc

V = 100000
MAX_SW = 8
D = 64
L = 16

C = 80
ROWS = C * MAX_SW
NCHUNK = V // C
GSZ = 128
NG = ROWS // GSZ


def _make_kernel(n_sub):
    info = plsc.get_sparse_core_info()
    nc, ns = info.num_cores, info.num_subcores
    nw = nc * ns
    iters = -(-NCHUNK // nw)
    zrow = n_sub
    assert nw * (iters - 1) <= NCHUNK and iters % 2 == 0

    mesh = plsc.VectorSubcoreMesh(core_axis_name="c", subcore_axis_name="s")

    @functools.partial(
        pl.kernel, mesh=mesh,
        compiler_params=pltpu.CompilerParams(
            use_tc_tiling_on_sc=False, needs_layout_passes=False),
        out_type=jax.ShapeDtypeStruct((V, D), jnp.float32),
        scratch_types=[
            pltpu.VMEM((ROWS,), jnp.int32),
            pltpu.VMEM((ROWS,), jnp.int32),
            pltpu.VMEM((C,), jnp.int32),
            pltpu.VMEM((C,), jnp.int32),
            pltpu.VMEM((C,), jnp.float32),
            pltpu.VMEM((C,), jnp.float32),
            pltpu.VMEM((ROWS, D), jnp.float32),
            pltpu.VMEM((ROWS, D), jnp.float32),
            pltpu.VMEM((C, D), jnp.float32),
            pltpu.VMEM((C, D), jnp.float32),
            pltpu.SemaphoreType.DMA,
            pltpu.SemaphoreType.DMA,
        ] + [pltpu.SemaphoreType.DMA] * 10,
    )
    def k(table, idxflat, lens, out, idx0, idx1, len0, len1, rec0, rec1,
          rows0, rows1, outc0, outc1, fsem0, fsem1, *gsems):
        wid = lax.axis_index("s") * nc + lax.axis_index("c")
        idx_s = (idx0, idx1)
        len_s = (len0, len1)
        rec_s = (rec0, rec1)
        rows_s = (rows0, rows1)
        outc_s = (outc0, outc1)
        fsem_s = (fsem0, fsem1)
        gsem_s = (gsems[:5], gsems[5:])
        nlast = jnp.int32(NCHUNK - 1)

        def fire_fetch(t):
            p = t if isinstance(t, int) else None
            sl = (t % 2) if isinstance(t, int) else None
            del p, sl

        def fetch(t, sl):
            ch = jnp.minimum(wid + t * nw, nlast)
            pltpu.async_copy(idxflat.at[pl.ds(ch * ROWS, ROWS)],
                             idx_s[sl], fsem_s[sl])
            pltpu.async_copy(lens.at[pl.ds(ch * C, C)], len_s[sl], fsem_s[sl])

        def wait_fetch(t, sl):
            ch = jnp.minimum(wid + t * nw, nlast)
            pltpu.make_async_copy(idxflat.at[pl.ds(ch * ROWS, ROWS)],
                                  idx_s[sl], fsem_s[sl]).wait()
            pltpu.make_async_copy(lens.at[pl.ds(ch * C, C)],
                                  len_s[sl], fsem_s[sl]).wait()

        def mask_rec(sl):
            idx_v, len_v, rec_v = idx_s[sl], len_s[sl], rec_s[sl]

            def mask_body(i, c2):
                jpat = lax.iota(jnp.int32, L) % MAX_SW
                wpat = lax.iota(jnp.int32, L) // MAX_SW
                zvec = lax.broadcast(jnp.int32(zrow), (L,))
                v = idx_v[pl.ds(i * L, L)]
                ln = plsc.load_gather(len_v, [i * 2 + wpat])
                idx_v[pl.ds(i * L, L)] = jnp.where(jpat < ln, v, zvec)
                return c2
            lax.fori_loop(0, ROWS // L, mask_body, 0)

            def rec_body(i, c2):
                lv = len_v[pl.ds(i * L, L)].astype(jnp.float32)
                rec_v[pl.ds(i * L, L)] = 1.0 / lv
                return c2
            lax.fori_loop(0, C // L, rec_body, 0)

        def fire_gathers(sl):
            for g in range(NG):
                pltpu.async_copy(
                    table.at[idx_s[sl].at[pl.ds(g * GSZ, GSZ)]],
                    rows_s[sl].at[pl.ds(g * GSZ, GSZ)], gsem_s[sl][g])

        def wait_gathers(sl):
            for g in range(NG):
                pltpu.make_async_copy(
                    table.at[idx_s[sl].at[pl.ds(g * GSZ, GSZ)]],
                    rows_s[sl].at[pl.ds(g * GSZ, GSZ)], gsem_s[sl][g]).wait()

        def reduce_out(t, sl, guard):
            rows_v, outc_v, rec_v = rows_s[sl], outc_s[sl], rec_s[sl]

            def red_body(c, c2):
                rb = c * MAX_SW
                rvec = plsc.load_gather(
                    rec_v, [lax.broadcast(c.astype(jnp.int32), (L,))])
                for kk in range(D // L):
                    acc = rows_v[rb, pl.ds(kk * L, L)]
                    for j in range(1, MAX_SW):
                        acc = acc + rows_v[rb + j, pl.ds(kk * L, L)]
                    outc_v[c, pl.ds(kk * L, L)] = acc * rvec
                return c2
            lax.fori_loop(0, C, red_body, 0)

            chunk = wid + t * nw
            if guard:
                @pl.when(chunk < NCHUNK)
                def _():
                    pltpu.sync_copy(outc_v, out.at[pl.ds(chunk * C, C)])
            else:
                pltpu.sync_copy(outc_v, out.at[pl.ds(chunk * C, C)])

        def step(t, sl, fire_g=True, fire_f=True, guard=False):
            if fire_g:
                wait_fetch(t + 1, 1 - sl)
                mask_rec(1 - sl)
            wait_gathers(sl)
            if fire_g:
                fire_gathers(1 - sl)
            if fire_f:
                fetch(t + 2, sl)
            reduce_out(t, sl, guard)

        fetch(jnp.int32(0), 0)
        wait_fetch(jnp.int32(0), 0)
        mask_rec(0)
        fire_gathers(0)
        fetch(jnp.int32(1), 1)

        def pair_body(u, carry):
            t = u * 2
            step(t, 0)
            step(t + 1, 1)
            return carry
        lax.fori_loop(0, (iters - 2) // 2, pair_body, 0)

        t_tail = jnp.int32(iters - 2)
        step(t_tail, 0, fire_f=False)
        step(t_tail + 1, 1, fire_g=False, fire_f=False, guard=True)

    return k


def kernel(sw_table, word2subword, word2subword_len):
    n_sub = sw_table.shape[0]
    table = jnp.concatenate(
        [sw_table.astype(jnp.float32), jnp.zeros((8, D), jnp.float32)], axis=0)
    idxflat = word2subword.astype(jnp.int32).reshape(-1)
    lens = word2subword_len.astype(jnp.int32)
    return _make_kernel(n_sub)(table, idxflat, lens)
